# baseline (speedup 1.0000x reference)
import math

import jax
import jax.numpy as jnp
from jax.experimental import pallas as pl
from jax.experimental.pallas import tpu as pltpu


def _net_kernel(ks, obs_ref, m_ref,
                W1_ref, b1_ref, Ws1_ref, bs1_ref,
                W2_ref, b2_ref, Ws2_ref, bs2_ref,
                W3_ref, b3_ref, Ws3_ref, bs3_ref,
                Wl_ref, bl_ref, out_ref):
    x = obs_ref[0]
    m = m_ref[0]
    a = x.shape[0]

    iota_col = jax.lax.broadcasted_iota(jnp.int32, (a, 1), 0)
    iota_row = jax.lax.broadcasted_iota(jnp.int32, (1, a), 1)
    is0 = iota_col == 0

    def gcn(xx, Wmat, bias_row, mm):
        h = jnp.dot(xx, Wmat, preferred_element_type=jnp.float32)
        m0 = mm[0:1, 0:1]
        S = jnp.sum(mm)
        deg = jnp.where(is0, m0 * (m0 + S + 1.0), mm * (1.0 + m0))
        deg_safe = jnp.where(deg > 0, deg, 1.0)
        dinv = jax.lax.rsqrt(deg_safe)
        dinv0 = dinv[0:1, 0:1]
        coef_hub = m0 * mm * dinv0 * dinv
        self_coef = mm / deg_safe
        h0 = h[0:1, :]
        hubrow = jnp.sum((mm * dinv) * h, axis=0, keepdims=True)
        agg = coef_hub * h0 + self_coef * h
        agg = agg + jnp.where(is0, m0 * dinv0 * hubrow, 0.0)
        return agg + bias_row

    layers = ((W1_ref, b1_ref, Ws1_ref, bs1_ref),
              (W2_ref, b2_ref, Ws2_ref, bs2_ref),
              (W3_ref, b3_ref, Ws3_ref, bs3_ref))

    total = None
    for (W_ref, b_ref, Ws_ref, bs_ref), k in zip(layers, ks):
        x = jnp.maximum(gcn(x, W_ref[...], b_ref[...], m), 0.0)
        score = gcn(x, Ws_ref[...], bs_ref[...], m)

        s = jnp.where(m > 0, score, -1e9)
        s_row = jnp.transpose(s)
        gt = (s_row > s) | ((s_row == s) & (iota_row < iota_col))
        rank = jnp.sum(gt.astype(jnp.float32), axis=1, keepdims=True)
        newmask = (rank < float(k)).astype(jnp.float32)

        x = x * jnp.tanh(score) * newmask
        m = newmask

        ssum = jnp.sum(x * m, axis=0, keepdims=True)
        cnt = jnp.maximum(jnp.sum(m), 1.0)
        gap = ssum / cnt
        gmp = jnp.max(jnp.where(m > 0, x, -1e9), axis=0, keepdims=True)
        out_l = jnp.concatenate([gmp, gap], axis=1)
        total = out_l if total is None else total + out_l

    final = jnp.dot(total, Wl_ref[...], preferred_element_type=jnp.float32)
    final = jnp.maximum(final + bl_ref[...], 0.0)
    out_ref[0] = final


def kernel(obs, is_alive, W1, b1, Ws1, bs1, W2, b2, Ws2, bs2,
           W3, b3, Ws3, bs3, Wl, bl):
    b, a, f = obs.shape
    nhid = W1.shape[1]

    ks = []
    k = a
    for _ in range(3):
        k = math.ceil(0.5 * k)
        ks.append(k)
    ks = tuple(ks)

    mask3 = is_alive.reshape(b, a, 1)
    b1r, b2r, b3r = b1.reshape(1, -1), b2.reshape(1, -1), b3.reshape(1, -1)
    bs1r, bs2r, bs3r = bs1.reshape(1, 1), bs2.reshape(1, 1), bs3.reshape(1, 1)
    blr = bl.reshape(1, -1)

    def fixed(shape):
        nd = len(shape)
        return pl.BlockSpec(shape, lambda g: (0,) * nd)

    grid = (b,)
    in_specs = [
        pl.BlockSpec((1, a, f), lambda g: (g, 0, 0)),
        pl.BlockSpec((1, a, 1), lambda g: (g, 0, 0)),
        fixed((f, nhid)), fixed((1, nhid)), fixed((nhid, 1)), fixed((1, 1)),
        fixed((nhid, nhid)), fixed((1, nhid)), fixed((nhid, 1)), fixed((1, 1)),
        fixed((nhid, nhid)), fixed((1, nhid)), fixed((nhid, 1)), fixed((1, 1)),
        fixed((2 * nhid, nhid)), fixed((1, nhid)),
    ]
    out_spec = pl.BlockSpec((1, 1, nhid), lambda g: (g, 0, 0))

    import functools
    body = functools.partial(_net_kernel, ks)

    out = pl.pallas_call(
        body,
        grid=grid,
        in_specs=in_specs,
        out_specs=out_spec,
        out_shape=jax.ShapeDtypeStruct((b, 1, nhid), jnp.float32),
        compiler_params=pltpu.CompilerParams(
            dimension_semantics=("parallel",),
        ),
    )(obs, mask3, W1, b1r, Ws1, bs1r, W2, b2r, Ws2, bs2r,
      W3, b3r, Ws3, bs3r, Wl, blr)
    return out.reshape(b, nhid)


# speedup vs baseline: 1.2836x; 1.2836x over previous
import functools
import math

import jax
import jax.numpy as jnp
from jax.experimental import pallas as pl
from jax.experimental.pallas import tpu as pltpu

_G = 2


def _net_kernel(ks, obs_ref, m_ref,
                W1_ref, b1_ref, Ws1_ref, bs1_ref,
                W2_ref, b2_ref, Ws2_ref, bs2_ref,
                W3_ref, b3_ref, Ws3_ref, bs3_ref,
                Wl_ref, bl_ref, out_ref):
    ng = obs_ref.shape[0]
    a = obs_ref.shape[1]
    gs = range(ng)

    iota_c = jax.lax.broadcasted_iota(jnp.int32, (a, 1), 0)
    iota_r = jax.lax.broadcasted_iota(jnp.int32, (1, a), 1)
    is0_c = iota_c == 0
    idxlt = iota_r < iota_c
    ones_r = jnp.ones((1, a), jnp.float32)
    ones_c = jnp.ones((a, 1), jnp.float32)

    def mm(p, q):
        return jnp.dot(p, q, preferred_element_type=jnp.float32)

    layers = ((W1_ref, b1_ref, Ws1_ref, bs1_ref),
              (W2_ref, b2_ref, Ws2_ref, bs2_ref),
              (W3_ref, b3_ref, Ws3_ref, bs3_ref))

    m_c = [m_ref[g] for g in gs]
    xs = [obs_ref[g] * m_c[g] for g in gs]
    S = [mm(ones_r, m_c[g]) for g in gs]
    totals = [None] * ng

    for li, ((W_ref, b_ref, Ws_ref, bs_ref), k) in enumerate(zip(layers, ks)):
        W = W_ref[...]
        brow = b_ref[...]
        Ws = Ws_ref[...]
        bs = bs_ref[...]

        m0 = [m_c[g][0:1, 0:1] for g in gs]
        deg0 = [m0[g] * (S[g] + 2.0) for g in gs]
        deg0s = [jnp.where(deg0[g] > 0, deg0[g], 1.0) for g in gs]
        dinv0 = [jax.lax.rsqrt(deg0s[g]) for g in gs]
        selfc0 = [m0[g] / deg0s[g] for g in gs]
        r1 = [jax.lax.rsqrt(1.0 + m0[g]) for g in gs]
        c1 = [r1[g] * r1[g] for g in gs]
        c2 = [m0[g] * dinv0[g] * r1[g] for g in gs]
        cc = [2.0 * selfc0[g] - c1[g] - c2[g] for g in gs]

        h = [mm(xs[g], W) for g in gs]
        sum_h = [mm(ones_r, h[g]) for g in gs]
        h0 = [h[g][0:1, :] for g in gs]
        hub_r = [r1[g] * sum_h[g] + (dinv0[g] - r1[g]) * h0[g] for g in gs]
        corr = [cc[g] * h0[g] + (m0[g] * dinv0[g]) * hub_r[g] for g in gs]
        rowadd = [c2[g] * h0[g] + brow for g in gs]
        agg = [c1[g] * h[g] + jnp.where(is0_c, rowadd[g] + corr[g], rowadd[g])
               for g in gs]
        xs = [jnp.maximum(agg[g], 0.0) for g in gs]

        hs_c = [mm(xs[g], Ws) for g in gs]
        sum_hs = [mm(ones_r, hs_c[g]) for g in gs]
        hs0 = [hs_c[g][0:1, 0:1] for g in gs]
        hubs = [r1[g] * sum_hs[g] + (dinv0[g] - r1[g]) * hs0[g] for g in gs]
        scorr = [cc[g] * hs0[g] + (m0[g] * dinv0[g]) * hubs[g] for g in gs]
        srow = [c2[g] * hs0[g] + bs for g in gs]
        score_c = [c1[g] * hs_c[g]
                   + jnp.where(is0_c, srow[g] + scorr[g], srow[g]) for g in gs]

        s_c = [jnp.where(m_c[g] > 0, score_c[g], -1e9) for g in gs]
        s_r = [jnp.transpose(s_c[g]) for g in gs]
        sel = [(s_r[g] > s_c[g]) | ((s_r[g] == s_c[g]) & idxlt) for g in gs]
        Nf = [jnp.where(sel[g], 1.0, 0.0) for g in gs]
        rank_c = [mm(Nf[g], ones_c) for g in gs]
        newm = [jnp.where(rank_c[g] < float(k), 1.0, 0.0) for g in gs]

        t_c = [jnp.tanh(score_c[g]) * newm[g] for g in gs]
        xs = [xs[g] * t_c[g] for g in gs]
        m_c = newm
        S = [jnp.full((1, 1), float(k), jnp.float32)] * ng

        gap = [mm(ones_r, xs[g]) * (1.0 / float(k)) for g in gs]
        xneg = [jnp.where(newm[g] > 0, xs[g], -1e9) for g in gs]
        gmp = [jnp.max(xneg[g], axis=0, keepdims=True) for g in gs]
        out_l = [jnp.concatenate([gmp[g], gap[g]], axis=1) for g in gs]
        totals = [out_l[g] if totals[g] is None else totals[g] + out_l[g]
                  for g in gs]

    Wl = Wl_ref[...]
    bl = bl_ref[...]
    for g in gs:
        final = jnp.maximum(mm(totals[g], Wl) + bl, 0.0)
        out_ref[g] = final


def kernel(obs, is_alive, W1, b1, Ws1, bs1, W2, b2, Ws2, bs2,
           W3, b3, Ws3, bs3, Wl, bl):
    b, a, f = obs.shape
    nhid = W1.shape[1]

    ks = []
    k = a
    for _ in range(3):
        k = math.ceil(0.5 * k)
        ks.append(k)
    ks = tuple(ks)

    mask3 = is_alive.reshape(b, a, 1)
    b1r, b2r, b3r = b1.reshape(1, -1), b2.reshape(1, -1), b3.reshape(1, -1)
    bs1r, bs2r, bs3r = bs1.reshape(1, 1), bs2.reshape(1, 1), bs3.reshape(1, 1)
    blr = bl.reshape(1, -1)

    def fixed(shape):
        nd = len(shape)
        return pl.BlockSpec(shape, lambda g: (0,) * nd)

    grid = (b // _G,)
    in_specs = [
        pl.BlockSpec((_G, a, f), lambda g: (g, 0, 0)),
        pl.BlockSpec((_G, a, 1), lambda g: (g, 0, 0)),
        fixed((f, nhid)), fixed((1, nhid)), fixed((nhid, 1)), fixed((1, 1)),
        fixed((nhid, nhid)), fixed((1, nhid)), fixed((nhid, 1)), fixed((1, 1)),
        fixed((nhid, nhid)), fixed((1, nhid)), fixed((nhid, 1)), fixed((1, 1)),
        fixed((2 * nhid, nhid)), fixed((1, nhid)),
    ]
    out_spec = pl.BlockSpec((_G, 1, nhid), lambda g: (g, 0, 0))

    body = functools.partial(_net_kernel, ks)

    out = pl.pallas_call(
        body,
        grid=grid,
        in_specs=in_specs,
        out_specs=out_spec,
        out_shape=jax.ShapeDtypeStruct((b, 1, nhid), jnp.float32),
        compiler_params=pltpu.CompilerParams(
            dimension_semantics=("parallel",),
        ),
    )(obs, mask3, W1, b1r, Ws1, bs1r, W2, b2r, Ws2, bs2r,
      W3, b3r, Ws3, bs3r, Wl, blr)
    return out.reshape(b, nhid)


# speedup vs baseline: 1.3592x; 1.0589x over previous
import functools
import math

import jax
import jax.numpy as jnp
from jax.experimental import pallas as pl
from jax.experimental.pallas import tpu as pltpu

_G = 2


def _net_kernel(ks, obs_ref, m_ref,
                W1_ref, b1_ref, Ws1_ref, bs1_ref,
                W2_ref, b2_ref, Ws2_ref, bs2_ref,
                W3_ref, b3_ref, Ws3_ref, bs3_ref,
                Wl_ref, bl_ref, out_ref):
    ng = obs_ref.shape[0]
    a = obs_ref.shape[1]
    gs = range(ng)

    iota_c = jax.lax.broadcasted_iota(jnp.int32, (a, 1), 0)
    iota_r = jax.lax.broadcasted_iota(jnp.int32, (1, a), 1)
    is0_c = iota_c == 0
    idxlt = iota_r < iota_c
    ones_r = jnp.ones((1, a), jnp.float32)
    ones_c = jnp.ones((a, 1), jnp.float32)

    def mm(p, q):
        return jnp.dot(p, q, preferred_element_type=jnp.float32)

    layers = ((W1_ref, b1_ref, Ws1_ref, bs1_ref),
              (W2_ref, b2_ref, Ws2_ref, bs2_ref),
              (W3_ref, b3_ref, Ws3_ref, bs3_ref))

    m_c = [m_ref[g] for g in gs]
    xs = [obs_ref[g] for g in gs]
    S = [mm(ones_r, m_c[g]) for g in gs]
    totals = [None] * ng

    for li, ((W_ref, b_ref, Ws_ref, bs_ref), k) in enumerate(zip(layers, ks)):
        W = W_ref[...]
        brow = b_ref[...]
        Ws = Ws_ref[...]
        bs = bs_ref[...]

        m0 = [m_c[g][0:1, 0:1] for g in gs]
        deg = [jnp.where(is0_c, m0[g] * (m0[g] + S[g] + 1.0),
                         m_c[g] * (1.0 + m0[g])) for g in gs]
        degs = [jnp.where(deg[g] > 0, deg[g], 1.0) for g in gs]
        dinv = [jax.lax.rsqrt(degs[g]) for g in gs]
        dinv0 = [dinv[g][0:1, 0:1] for g in gs]
        selfc = [m_c[g] / degs[g] for g in gs]
        w_c = [m_c[g] * dinv[g] for g in gs]
        hubc = [m0[g] * dinv0[g] for g in gs]
        coefh = [hubc[g] * w_c[g] for g in gs]

        h = [mm(xs[g], W) for g in gs]
        h0 = [h[g][0:1, :] for g in gs]
        hubrow = [jnp.sum(w_c[g] * h[g], axis=0, keepdims=True) for g in gs]
        agg = [coefh[g] * h0[g] + selfc[g] * h[g] for g in gs]
        agg = [agg[g] + jnp.where(is0_c, hubc[g] * hubrow[g], 0.0) + brow
               for g in gs]
        xs = [jnp.maximum(agg[g], 0.0) for g in gs]

        hs_c = [mm(xs[g], Ws) for g in gs]
        hs0 = [hs_c[g][0:1, 0:1] for g in gs]
        hubs = [jnp.sum(w_c[g] * hs_c[g], axis=0, keepdims=True) for g in gs]
        score_c = [coefh[g] * hs0[g] + selfc[g] * hs_c[g] for g in gs]
        score_c = [score_c[g] + jnp.where(is0_c, hubc[g] * hubs[g], 0.0) + bs
                   for g in gs]

        s_c = [jnp.where(m_c[g] > 0, score_c[g], -1e9) for g in gs]
        s_r = [jnp.transpose(s_c[g]) for g in gs]
        sel = [(s_r[g] > s_c[g]) | ((s_r[g] == s_c[g]) & idxlt) for g in gs]
        Nf = [jnp.where(sel[g], 1.0, 0.0) for g in gs]
        rank_c = [mm(Nf[g], ones_c) for g in gs]
        newm = [jnp.where(rank_c[g] < float(k), 1.0, 0.0) for g in gs]

        t_c = [jnp.tanh(score_c[g]) * newm[g] for g in gs]
        xs = [xs[g] * t_c[g] for g in gs]
        m_c = newm
        S = [jnp.full((1, 1), float(k), jnp.float32)] * ng

        gap = [mm(ones_r, xs[g]) * (1.0 / float(k)) for g in gs]
        xneg = [jnp.where(newm[g] > 0, xs[g], -1e9) for g in gs]
        gmp = [jnp.max(xneg[g], axis=0, keepdims=True) for g in gs]
        out_l = [jnp.concatenate([gmp[g], gap[g]], axis=1) for g in gs]
        totals = [out_l[g] if totals[g] is None else totals[g] + out_l[g]
                  for g in gs]

    Wl = Wl_ref[...]
    bl = bl_ref[...]
    for g in gs:
        final = jnp.maximum(mm(totals[g], Wl) + bl, 0.0)
        out_ref[g] = final


def kernel(obs, is_alive, W1, b1, Ws1, bs1, W2, b2, Ws2, bs2,
           W3, b3, Ws3, bs3, Wl, bl):
    b, a, f = obs.shape
    nhid = W1.shape[1]

    ks = []
    k = a
    for _ in range(3):
        k = math.ceil(0.5 * k)
        ks.append(k)
    ks = tuple(ks)

    mask3 = is_alive.reshape(b, a, 1)
    b1r, b2r, b3r = b1.reshape(1, -1), b2.reshape(1, -1), b3.reshape(1, -1)
    bs1r, bs2r, bs3r = bs1.reshape(1, 1), bs2.reshape(1, 1), bs3.reshape(1, 1)
    blr = bl.reshape(1, -1)

    def fixed(shape):
        nd = len(shape)
        return pl.BlockSpec(shape, lambda g: (0,) * nd)

    grid = (b // _G,)
    in_specs = [
        pl.BlockSpec((_G, a, f), lambda g: (g, 0, 0)),
        pl.BlockSpec((_G, a, 1), lambda g: (g, 0, 0)),
        fixed((f, nhid)), fixed((1, nhid)), fixed((nhid, 1)), fixed((1, 1)),
        fixed((nhid, nhid)), fixed((1, nhid)), fixed((nhid, 1)), fixed((1, 1)),
        fixed((nhid, nhid)), fixed((1, nhid)), fixed((nhid, 1)), fixed((1, 1)),
        fixed((2 * nhid, nhid)), fixed((1, nhid)),
    ]
    out_spec = pl.BlockSpec((_G, 1, nhid), lambda g: (g, 0, 0))

    body = functools.partial(_net_kernel, ks)

    out = pl.pallas_call(
        body,
        grid=grid,
        in_specs=in_specs,
        out_specs=out_spec,
        out_shape=jax.ShapeDtypeStruct((b, 1, nhid), jnp.float32),
        compiler_params=pltpu.CompilerParams(
            dimension_semantics=("parallel",),
        ),
    )(obs, mask3, W1, b1r, Ws1, bs1r, W2, b2r, Ws2, bs2r,
      W3, b3r, Ws3, bs3r, Wl, blr)
    return out.reshape(b, nhid)


# speedup vs baseline: 1.6033x; 1.1796x over previous
import functools
import math

import jax
import jax.numpy as jnp
from jax.experimental import pallas as pl
from jax.experimental.pallas import tpu as pltpu

_G = 4


def _net_kernel(ks, obs_ref, m_ref,
                W1_ref, b1_ref, Ws1_ref, bs1_ref,
                W2_ref, b2_ref, Ws2_ref, bs2_ref,
                W3_ref, b3_ref, Ws3_ref, bs3_ref,
                Wl_ref, bl_ref, out_ref):
    ng = obs_ref.shape[0]
    a = obs_ref.shape[1]
    gs = range(ng)

    iota_c = jax.lax.broadcasted_iota(jnp.int32, (a, 1), 0)
    iota_r = jax.lax.broadcasted_iota(jnp.int32, (1, a), 1)
    is0_c = iota_c == 0
    idxlt = iota_r < iota_c
    ones_r = jnp.ones((1, a), jnp.float32)
    ones_c = jnp.ones((a, 1), jnp.float32)

    def mm(p, q):
        return jnp.dot(p, q, preferred_element_type=jnp.float32)

    layers = ((W1_ref, b1_ref, Ws1_ref, bs1_ref),
              (W2_ref, b2_ref, Ws2_ref, bs2_ref),
              (W3_ref, b3_ref, Ws3_ref, bs3_ref))

    m_c = [m_ref[g] for g in gs]
    xs = [obs_ref[g] for g in gs]
    S = [mm(ones_r, m_c[g]) for g in gs]
    totals = [None] * ng

    for li, ((W_ref, b_ref, Ws_ref, bs_ref), k) in enumerate(zip(layers, ks)):
        W = W_ref[...]
        brow = b_ref[...]
        Ws = Ws_ref[...]
        bs = bs_ref[...]

        m0 = [m_c[g][0:1, 0:1] for g in gs]
        deg = [jnp.where(is0_c, m0[g] * (m0[g] + S[g] + 1.0),
                         m_c[g] * (1.0 + m0[g])) for g in gs]
        degs = [jnp.where(deg[g] > 0, deg[g], 1.0) for g in gs]
        dinv = [jax.lax.rsqrt(degs[g]) for g in gs]
        dinv0 = [dinv[g][0:1, 0:1] for g in gs]
        selfc = [m_c[g] / degs[g] for g in gs]
        w_c = [m_c[g] * dinv[g] for g in gs]
        hubc = [m0[g] * dinv0[g] for g in gs]
        coefh = [hubc[g] * w_c[g] for g in gs]

        h = [mm(xs[g], W) for g in gs]
        h0 = [h[g][0:1, :] for g in gs]
        hubrow = [jnp.sum(w_c[g] * h[g], axis=0, keepdims=True) for g in gs]
        agg = [coefh[g] * h0[g] + selfc[g] * h[g] for g in gs]
        agg = [agg[g] + jnp.where(is0_c, hubc[g] * hubrow[g], 0.0) + brow
               for g in gs]
        xs = [jnp.maximum(agg[g], 0.0) for g in gs]

        hs_c = [mm(xs[g], Ws) for g in gs]
        hs0 = [hs_c[g][0:1, 0:1] for g in gs]
        hubs = [jnp.sum(w_c[g] * hs_c[g], axis=0, keepdims=True) for g in gs]
        score_c = [coefh[g] * hs0[g] + selfc[g] * hs_c[g] for g in gs]
        score_c = [score_c[g] + jnp.where(is0_c, hubc[g] * hubs[g], 0.0) + bs
                   for g in gs]

        s_c = [jnp.where(m_c[g] > 0, score_c[g], -1e9) for g in gs]
        s_r = [jnp.transpose(s_c[g]) for g in gs]
        sel = [(s_r[g] > s_c[g]) | ((s_r[g] == s_c[g]) & idxlt) for g in gs]
        Nf = [jnp.where(sel[g], 1.0, 0.0) for g in gs]
        rank_c = [mm(Nf[g], ones_c) for g in gs]
        newm = [jnp.where(rank_c[g] < float(k), 1.0, 0.0) for g in gs]

        t_c = [jnp.tanh(score_c[g]) * newm[g] for g in gs]
        xs = [xs[g] * t_c[g] for g in gs]
        m_c = newm
        S = [jnp.full((1, 1), float(k), jnp.float32)] * ng

        gap = [mm(ones_r, xs[g]) * (1.0 / float(k)) for g in gs]
        xneg = [jnp.where(newm[g] > 0, xs[g], -1e9) for g in gs]
        gmp = [jnp.max(xneg[g], axis=0, keepdims=True) for g in gs]
        out_l = [jnp.concatenate([gmp[g], gap[g]], axis=1) for g in gs]
        totals = [out_l[g] if totals[g] is None else totals[g] + out_l[g]
                  for g in gs]

    Wl = Wl_ref[...]
    bl = bl_ref[...]
    for g in gs:
        final = jnp.maximum(mm(totals[g], Wl) + bl, 0.0)
        out_ref[g] = final


def kernel(obs, is_alive, W1, b1, Ws1, bs1, W2, b2, Ws2, bs2,
           W3, b3, Ws3, bs3, Wl, bl):
    b, a, f = obs.shape
    nhid = W1.shape[1]

    ks = []
    k = a
    for _ in range(3):
        k = math.ceil(0.5 * k)
        ks.append(k)
    ks = tuple(ks)

    mask3 = is_alive.reshape(b, a, 1)
    b1r, b2r, b3r = b1.reshape(1, -1), b2.reshape(1, -1), b3.reshape(1, -1)
    bs1r, bs2r, bs3r = bs1.reshape(1, 1), bs2.reshape(1, 1), bs3.reshape(1, 1)
    blr = bl.reshape(1, -1)

    def fixed(shape):
        nd = len(shape)
        return pl.BlockSpec(shape, lambda g: (0,) * nd)

    grid = (b // _G,)
    in_specs = [
        pl.BlockSpec((_G, a, f), lambda g: (g, 0, 0)),
        pl.BlockSpec((_G, a, 1), lambda g: (g, 0, 0)),
        fixed((f, nhid)), fixed((1, nhid)), fixed((nhid, 1)), fixed((1, 1)),
        fixed((nhid, nhid)), fixed((1, nhid)), fixed((nhid, 1)), fixed((1, 1)),
        fixed((nhid, nhid)), fixed((1, nhid)), fixed((nhid, 1)), fixed((1, 1)),
        fixed((2 * nhid, nhid)), fixed((1, nhid)),
    ]
    out_spec = pl.BlockSpec((_G, 1, nhid), lambda g: (g, 0, 0))

    body = functools.partial(_net_kernel, ks)

    out = pl.pallas_call(
        body,
        grid=grid,
        in_specs=in_specs,
        out_specs=out_spec,
        out_shape=jax.ShapeDtypeStruct((b, 1, nhid), jnp.float32),
        compiler_params=pltpu.CompilerParams(
            dimension_semantics=("parallel",),
        ),
    )(obs, mask3, W1, b1r, Ws1, bs1r, W2, b2r, Ws2, bs2r,
      W3, b3r, Ws3, bs3r, Wl, blr)
    return out.reshape(b, nhid)
